# baseline (speedup 1.0000x reference)
import functools

import jax
import jax.numpy as jnp
from jax import lax
from jax.experimental import pallas as pl
from jax.experimental.pallas import tpu as pltpu
from jax.experimental.pallas import tpu_sc as plsc

_EPS = 0.001
_NC = 19
_NBINS = 384
_HW = _NBINS * 16
_CH = 2048


def _make_k1(bsz, npix):
    info = plsc.get_sparse_core_info()
    nw = info.num_cores * info.num_subcores
    tiles_per_b = nw // bsz
    per_tile = npix // tiles_per_b
    nsteps = per_tile // _CH
    mesh = plsc.VectorSubcoreMesh(core_axis_name="c", subcore_axis_name="s")

    @functools.partial(
        pl.kernel, mesh=mesh,
        out_type=jax.ShapeDtypeStruct((nw * _HW,), jnp.float32),
        compiler_params=pltpu.CompilerParams(needs_layout_passes=False),
        scratch_types=[
            pltpu.VMEM((2 * _NC * _CH,), jnp.float32),
            pltpu.VMEM((2 * _CH,), jnp.int32),
            pltpu.VMEM((_HW,), jnp.float32),
            pltpu.SemaphoreType.DMA,
            pltpu.SemaphoreType.DMA,
        ],
    )
    def k1(pred_hbm, true_hbm, out_hbm, pbuf, tbuf, hist, sem0, sem1):
        wid = lax.axis_index("s") * info.num_cores + lax.axis_index("c")
        b = wid // tiles_per_b
        pix0 = (wid % tiles_per_b) * per_tile
        sems = (sem0, sem1)

        def zero_hist(r, carry):
            hist[pl.ds(r * 16, 16)] = jnp.zeros((16,), jnp.float32)
            return carry
        lax.fori_loop(0, _NBINS, zero_hist, 0)

        def copies(s, slot):
            cps = []
            for c in range(_NC):
                off = (b * _NC + c) * npix + pix0 + s * _CH
                cps.append(pltpu.make_async_copy(
                    pred_hbm.at[pl.ds(off, _CH)],
                    pbuf.at[pl.ds((slot * _NC + c) * _CH, _CH)],
                    sems[slot]))
            cps.append(pltpu.make_async_copy(
                true_hbm.at[pl.ds(b * npix + pix0 + s * _CH, _CH)],
                tbuf.at[pl.ds(slot * _CH, _CH)], sems[slot]))
            return cps

        def start(s, slot):
            for cp in copies(s, slot):
                cp.start()

        def wait(s, slot):
            for cp in copies(s, slot):
                cp.wait()

        start(0, 0)
        lane = lax.iota(jnp.int32, 16)
        ones = jnp.ones((16,), jnp.float32)

        def process(slot):
            def pix(i, c2):
                base = slot * _NC * _CH + i * 16
                v = pbuf[pl.ds(base, 16)]
                idx = jnp.zeros((16,), jnp.int32)
                for c in range(1, _NC):
                    xc = pbuf[pl.ds(base + c * _CH, 16)]
                    m = xc > v
                    v = jnp.where(m, xc, v)
                    idx = jnp.where(m, jnp.full((16,), c, jnp.int32), idx)
                t = tbuf[pl.ds(slot * _CH + i * 16, 16)]
                bins = (idx * _NC + t) * 16 + lane
                plsc.addupdate_scatter(hist, [bins], ones)
                return c2

            lax.fori_loop(0, _CH // 16, pix, 0)

        def pair(j, carry):
            s0 = 2 * j
            s1 = s0 + 1
            start(s1, 1)
            wait(s0, 0)
            process(0)
            @pl.when(s1 + 1 < nsteps)
            def _():
                start(s1 + 1, 0)
            wait(s1, 1)
            process(1)
            return carry

        lax.fori_loop(0, nsteps // 2, pair, 0)
        pltpu.sync_copy(hist, out_hbm.at[pl.ds(wid * _HW, _HW)])

    return k1


def _make_k2(nw):
    info = plsc.get_sparse_core_info()
    mesh = plsc.VectorSubcoreMesh(core_axis_name="c", subcore_axis_name="s")

    @functools.partial(
        pl.kernel, mesh=mesh,
        out_type=jax.ShapeDtypeStruct((16,), jnp.float32),
        compiler_params=pltpu.CompilerParams(needs_layout_passes=False),
        scratch_types=[
            pltpu.VMEM((_HW,), jnp.float32),
            pltpu.VMEM((_HW,), jnp.float32),
            pltpu.VMEM((16,), jnp.float32),
        ],
    )
    def k2(hists_hbm, out_hbm, acc, tmp, obuf):
        wid = lax.axis_index("s") * info.num_cores + lax.axis_index("c")

        @pl.when(wid == 0)
        def _():
            pltpu.sync_copy(hists_hbm.at[pl.ds(0, _HW)], acc)

            def add_src(w, carry):
                pltpu.sync_copy(hists_hbm.at[pl.ds(w * _HW, _HW)], tmp)

                def add_row(r, c2):
                    sl = pl.ds(r * 16, 16)
                    acc[sl] = acc[sl] + tmp[sl]
                    return c2
                lax.fori_loop(0, _NBINS, add_row, 0)
                return carry
            lax.fori_loop(1, nw, add_src, 0)

            lane = lax.iota(jnp.int32, 16)
            zero = jnp.zeros((16,), jnp.float32)

            def accum(bin_, carry):
                a1, a2, b1, b2, d1, d2 = carry
                p = bin_ // _NC
                t = bin_ - p * _NC
                s = lax.reduce_sum(acc[pl.ds(bin_ * 16, 16)], axes=(0,))
                sv = jnp.full((16,), s, jnp.float32)
                pv = jnp.full((16,), p, jnp.int32)
                tv = jnp.full((16,), t, jnp.int32)
                a1 = a1 + jnp.where(lane == pv, sv, zero)
                a2 = a2 + jnp.where(lane == pv - 16, sv, zero)
                b1 = b1 + jnp.where(lane == tv, sv, zero)
                b2 = b2 + jnp.where(lane == tv - 16, sv, zero)
                dv = jnp.where(pv == tv, sv, zero)
                d1 = d1 + jnp.where(lane == pv, dv, zero)
                d2 = d2 + jnp.where(lane == pv - 16, dv, zero)
                return (a1, a2, b1, b2, d1, d2)

            init = (zero, zero, zero, zero, zero, zero)
            a1, a2, b1, b2, d1, d2 = lax.fori_loop(0, _NC * _NC, accum, init)

            dice1 = 2.0 * d1 / (a1 + b1 + _EPS)
            dice2 = 2.0 * d2 / (a2 + b2 + _EPS)
            jac1 = d1 / (a1 + b1 - d1 + _EPS)
            jac2 = d2 / (a2 + b2 - d2 + _EPS)
            dm = (jnp.full((16,), lax.reduce_sum(dice1, axes=(0,)), jnp.float32)
                  + jnp.full((16,), lax.reduce_sum(dice2, axes=(0,)), jnp.float32)
                  ) / float(_NC)
            jm = (jnp.full((16,), lax.reduce_sum(jac1, axes=(0,)), jnp.float32)
                  + jnp.full((16,), lax.reduce_sum(jac2, axes=(0,)), jnp.float32)
                  ) / float(_NC)
            obuf[...] = 1.0 - (dm + jm) / 2.0
            pltpu.sync_copy(obuf, out_hbm)

    return k2


@jax.jit
def _run(pred, true):
    bsz, nc, h, w = pred.shape
    npix = h * w
    pred2 = pred.reshape(-1)
    true2 = true.reshape(-1).astype(jnp.int32)
    info = plsc.get_sparse_core_info()
    nw = info.num_cores * info.num_subcores
    hists = _make_k1(bsz, npix)(pred2, true2)
    out = _make_k2(nw)(hists)
    return out[0]


def kernel(pred, true):
    return _run(pred, true)


# speedup vs baseline: 1.0080x; 1.0080x over previous
import functools

import jax
import jax.numpy as jnp
from jax import lax
from jax.experimental import pallas as pl
from jax.experimental.pallas import tpu as pltpu
from jax.experimental.pallas import tpu_sc as plsc

_EPS = 0.001
_NC = 19
_NBINS = 384
_HW = _NBINS * 16
_CH = 2048


def _make_k1(bsz, npix):
    info = plsc.get_sparse_core_info()
    nw = info.num_cores * info.num_subcores
    tiles_per_b = nw // bsz
    per_tile = npix // tiles_per_b
    nsteps = per_tile // _CH
    mesh = plsc.VectorSubcoreMesh(core_axis_name="c", subcore_axis_name="s")

    @functools.partial(
        pl.kernel, mesh=mesh,
        out_type=jax.ShapeDtypeStruct((nw * _HW,), jnp.float32),
        compiler_params=pltpu.CompilerParams(
            needs_layout_passes=False, use_tc_tiling_on_sc=False),
        scratch_types=[
            pltpu.VMEM((2, _NC, _CH), jnp.float32),
            pltpu.VMEM((2, _CH), jnp.int32),
            pltpu.VMEM((_HW,), jnp.float32),
            pltpu.SemaphoreType.DMA,
            pltpu.SemaphoreType.DMA,
        ],
    )
    def k1(pred_hbm, true_hbm, out_hbm, pbuf, tbuf, hist, sem0, sem1):
        wid = lax.axis_index("s") * info.num_cores + lax.axis_index("c")
        b = wid // tiles_per_b
        pix0 = (wid % tiles_per_b) * per_tile
        sems = (sem0, sem1)

        def zero_hist(r, carry):
            hist[pl.ds(r * 16, 16)] = jnp.zeros((16,), jnp.float32)
            return carry
        lax.fori_loop(0, _NBINS, zero_hist, 0)

        def copies(s, slot):
            return [
                pltpu.make_async_copy(
                    pred_hbm.at[b, :, pl.ds(pix0 + s * _CH, _CH)],
                    pbuf.at[slot], sems[slot]),
                pltpu.make_async_copy(
                    true_hbm.at[pl.ds(b * npix + pix0 + s * _CH, _CH)],
                    tbuf.at[slot], sems[slot]),
            ]

        def start(s, slot):
            for cp in copies(s, slot):
                cp.start()

        def wait(s, slot):
            for cp in copies(s, slot):
                cp.wait()

        start(0, 0)
        lane = lax.iota(jnp.int32, 16)
        ones = jnp.ones((16,), jnp.float32)

        def process(slot):
            def pix(i, c2):
                sl = pl.ds(i * 16, 16)
                v = pbuf[slot, 0, sl]
                idx = jnp.zeros((16,), jnp.int32)
                for c in range(1, _NC):
                    xc = pbuf[slot, c, sl]
                    m = xc > v
                    v = jnp.where(m, xc, v)
                    idx = jnp.where(m, jnp.full((16,), c, jnp.int32), idx)
                t = tbuf[slot, sl]
                bins = (idx * _NC + t) * 16 + lane
                plsc.addupdate_scatter(hist, [bins], ones)
                return c2

            lax.fori_loop(0, _CH // 16, pix, 0)

        def pair(j, carry):
            s0 = 2 * j
            s1 = s0 + 1
            start(s1, 1)
            wait(s0, 0)
            process(0)
            @pl.when(s1 + 1 < nsteps)
            def _():
                start(s1 + 1, 0)
            wait(s1, 1)
            process(1)
            return carry

        lax.fori_loop(0, nsteps // 2, pair, 0)
        pltpu.sync_copy(hist, out_hbm.at[pl.ds(wid * _HW, _HW)])

    return k1


def _make_k2(nw):
    info = plsc.get_sparse_core_info()
    mesh = plsc.VectorSubcoreMesh(core_axis_name="c", subcore_axis_name="s")

    @functools.partial(
        pl.kernel, mesh=mesh,
        out_type=jax.ShapeDtypeStruct((16,), jnp.float32),
        compiler_params=pltpu.CompilerParams(needs_layout_passes=False),
        scratch_types=[
            pltpu.VMEM((_HW,), jnp.float32),
            pltpu.VMEM((_HW,), jnp.float32),
            pltpu.VMEM((16,), jnp.float32),
        ],
    )
    def k2(hists_hbm, out_hbm, acc, tmp, obuf):
        wid = lax.axis_index("s") * info.num_cores + lax.axis_index("c")

        @pl.when(wid == 0)
        def _():
            pltpu.sync_copy(hists_hbm.at[pl.ds(0, _HW)], acc)

            def add_src(w, carry):
                pltpu.sync_copy(hists_hbm.at[pl.ds(w * _HW, _HW)], tmp)

                def add_row(r, c2):
                    sl = pl.ds(r * 16, 16)
                    acc[sl] = acc[sl] + tmp[sl]
                    return c2
                lax.fori_loop(0, _NBINS, add_row, 0)
                return carry
            lax.fori_loop(1, nw, add_src, 0)

            lane = lax.iota(jnp.int32, 16)
            zero = jnp.zeros((16,), jnp.float32)

            def accum(bin_, carry):
                a1, a2, b1, b2, d1, d2 = carry
                p = bin_ // _NC
                t = bin_ - p * _NC
                s = lax.reduce_sum(acc[pl.ds(bin_ * 16, 16)], axes=(0,))
                sv = jnp.full((16,), s, jnp.float32)
                pv = jnp.full((16,), p, jnp.int32)
                tv = jnp.full((16,), t, jnp.int32)
                a1 = a1 + jnp.where(lane == pv, sv, zero)
                a2 = a2 + jnp.where(lane == pv - 16, sv, zero)
                b1 = b1 + jnp.where(lane == tv, sv, zero)
                b2 = b2 + jnp.where(lane == tv - 16, sv, zero)
                dv = jnp.where(pv == tv, sv, zero)
                d1 = d1 + jnp.where(lane == pv, dv, zero)
                d2 = d2 + jnp.where(lane == pv - 16, dv, zero)
                return (a1, a2, b1, b2, d1, d2)

            init = (zero, zero, zero, zero, zero, zero)
            a1, a2, b1, b2, d1, d2 = lax.fori_loop(0, _NC * _NC, accum, init)

            dice1 = 2.0 * d1 / (a1 + b1 + _EPS)
            dice2 = 2.0 * d2 / (a2 + b2 + _EPS)
            jac1 = d1 / (a1 + b1 - d1 + _EPS)
            jac2 = d2 / (a2 + b2 - d2 + _EPS)
            dm = (jnp.full((16,), lax.reduce_sum(dice1, axes=(0,)), jnp.float32)
                  + jnp.full((16,), lax.reduce_sum(dice2, axes=(0,)), jnp.float32)
                  ) / float(_NC)
            jm = (jnp.full((16,), lax.reduce_sum(jac1, axes=(0,)), jnp.float32)
                  + jnp.full((16,), lax.reduce_sum(jac2, axes=(0,)), jnp.float32)
                  ) / float(_NC)
            obuf[...] = 1.0 - (dm + jm) / 2.0
            pltpu.sync_copy(obuf, out_hbm)

    return k2


@jax.jit
def _run(pred, true):
    bsz, nc, h, w = pred.shape
    npix = h * w
    pred2 = pred.reshape(bsz, nc, npix)
    true2 = true.reshape(-1).astype(jnp.int32)
    info = plsc.get_sparse_core_info()
    nw = info.num_cores * info.num_subcores
    hists = _make_k1(bsz, npix)(pred2, true2)
    out = _make_k2(nw)(hists)
    return out[0]


def kernel(pred, true):
    return _run(pred, true)


# speedup vs baseline: 1.2124x; 1.2028x over previous
import functools

import jax
import jax.numpy as jnp
from jax import lax
from jax.experimental import pallas as pl
from jax.experimental.pallas import tpu as pltpu
from jax.experimental.pallas import tpu_sc as plsc

_EPS = 0.001
_NC = 19
_NBINS = 384
_HW = _NBINS * 16
_CH = 2048


def _make_k1(bsz, npix):
    info = plsc.get_sparse_core_info()
    nw = info.num_cores * info.num_subcores
    tiles_per_b = nw // bsz
    per_tile = npix // tiles_per_b
    nsteps = per_tile // _CH
    mesh = plsc.VectorSubcoreMesh(core_axis_name="c", subcore_axis_name="s")

    @functools.partial(
        pl.kernel, mesh=mesh,
        out_type=jax.ShapeDtypeStruct((nw * _HW,), jnp.float32),
        compiler_params=pltpu.CompilerParams(
            needs_layout_passes=False, use_tc_tiling_on_sc=False),
        scratch_types=[
            pltpu.VMEM((2, _NC, _CH), jnp.float32),
            pltpu.VMEM((2, _CH), jnp.int32),
            pltpu.VMEM((_HW,), jnp.float32),
            pltpu.SemaphoreType.DMA,
            pltpu.SemaphoreType.DMA,
        ],
    )
    def k1(pred_hbm, true_hbm, out_hbm, pbuf, tbuf, hist, sem0, sem1):
        wid = lax.axis_index("s") * info.num_cores + lax.axis_index("c")
        b = wid // tiles_per_b
        pix0 = (wid % tiles_per_b) * per_tile
        sems = (sem0, sem1)

        def zero_hist(r, carry):
            hist[pl.ds(r * 16, 16)] = jnp.zeros((16,), jnp.float32)
            return carry
        lax.fori_loop(0, _NBINS, zero_hist, 0)

        def copies(s, slot):
            return [
                pltpu.make_async_copy(
                    pred_hbm.at[b, :, pl.ds(pix0 + s * _CH, _CH)],
                    pbuf.at[slot], sems[slot]),
                pltpu.make_async_copy(
                    true_hbm.at[pl.ds(b * npix + pix0 + s * _CH, _CH)],
                    tbuf.at[slot], sems[slot]),
            ]

        def start(s, slot):
            for cp in copies(s, slot):
                cp.start()

        def wait(s, slot):
            for cp in copies(s, slot):
                cp.wait()

        start(0, 0)
        lane = lax.iota(jnp.int32, 16)
        ones = jnp.ones((16,), jnp.float32)

        def process(slot):
            def pix(i, c2):
                sl = pl.ds(i * 16, 16)
                v = pbuf[slot, 0, sl]
                idx = jnp.zeros((16,), jnp.int32)
                for c in range(1, _NC):
                    xc = pbuf[slot, c, sl]
                    m = xc > v
                    v = jnp.where(m, xc, v)
                    idx = jnp.where(m, jnp.full((16,), c, jnp.int32), idx)
                t = tbuf[slot, sl]
                bins = (idx * _NC + t) * 16 + lane
                plsc.addupdate_scatter(hist, [bins], ones)
                return c2

            def pix4(q, c2):
                for u in range(4):
                    pix(q * 4 + u, c2)
                return c2

            lax.fori_loop(0, _CH // 64, pix4, 0)

        def pair(j, carry):
            s0 = 2 * j
            s1 = s0 + 1
            start(s1, 1)
            wait(s0, 0)
            process(0)
            @pl.when(s1 + 1 < nsteps)
            def _():
                start(s1 + 1, 0)
            wait(s1, 1)
            process(1)
            return carry

        lax.fori_loop(0, nsteps // 2, pair, 0)
        pltpu.sync_copy(hist, out_hbm.at[pl.ds(wid * _HW, _HW)])

    return k1


def _make_k2(nw):
    info = plsc.get_sparse_core_info()
    mesh = plsc.VectorSubcoreMesh(core_axis_name="c", subcore_axis_name="s")

    @functools.partial(
        pl.kernel, mesh=mesh,
        out_type=jax.ShapeDtypeStruct((16,), jnp.float32),
        compiler_params=pltpu.CompilerParams(needs_layout_passes=False),
        scratch_types=[
            pltpu.VMEM((_HW,), jnp.float32),
            pltpu.VMEM((2 * _HW,), jnp.float32),
            pltpu.VMEM((16,), jnp.float32),
            pltpu.SemaphoreType.DMA,
        ],
    )
    def k2(hists_hbm, out_hbm, acc, tmp, obuf, sem):
        wid = lax.axis_index("s") * info.num_cores + lax.axis_index("c")

        @pl.when(wid == 0)
        def _():
            pltpu.sync_copy(hists_hbm.at[pl.ds(0, _HW)], acc)

            def cp_src(w, slot):
                return pltpu.make_async_copy(
                    hists_hbm.at[pl.ds(w * _HW, _HW)],
                    tmp.at[pl.ds(slot * _HW, _HW)], sem)

            cp_src(1, 0).start()

            def add_src(j, carry):
                w = 1 + 2 * j

                def do(w_, slot):
                    @pl.when(w_ + 1 < nw)
                    def _():
                        cp_src(w_ + 1, 1 - slot).start()
                    cp_src(w_, slot).wait()

                    def add_row(r, c2):
                        for u in range(8):
                            sl = pl.ds((r * 8 + u) * 16, 16)
                            acc[sl] = acc[sl] + tmp[pl.ds(slot * _HW + (r * 8 + u) * 16, 16)]
                        return c2
                    lax.fori_loop(0, _NBINS // 8, add_row, 0)

                do(w, 0)
                @pl.when(w + 1 < nw)
                def _():
                    do(w + 1, 1)
                return carry
            lax.fori_loop(0, (nw - 1 + 1) // 2, add_src, 0)

            lane = lax.iota(jnp.int32, 16)
            zero = jnp.zeros((16,), jnp.float32)

            def accum(bin_, carry):
                a1, a2, b1, b2, d1, d2 = carry
                p = bin_ // _NC
                t = bin_ - p * _NC
                s = lax.reduce_sum(acc[pl.ds(bin_ * 16, 16)], axes=(0,))
                sv = jnp.full((16,), s, jnp.float32)
                pv = jnp.full((16,), p, jnp.int32)
                tv = jnp.full((16,), t, jnp.int32)
                a1 = a1 + jnp.where(lane == pv, sv, zero)
                a2 = a2 + jnp.where(lane == pv - 16, sv, zero)
                b1 = b1 + jnp.where(lane == tv, sv, zero)
                b2 = b2 + jnp.where(lane == tv - 16, sv, zero)
                dv = jnp.where(pv == tv, sv, zero)
                d1 = d1 + jnp.where(lane == pv, dv, zero)
                d2 = d2 + jnp.where(lane == pv - 16, dv, zero)
                return (a1, a2, b1, b2, d1, d2)

            init = (zero, zero, zero, zero, zero, zero)
            a1, a2, b1, b2, d1, d2 = lax.fori_loop(0, _NC * _NC, accum, init)

            dice1 = 2.0 * d1 / (a1 + b1 + _EPS)
            dice2 = 2.0 * d2 / (a2 + b2 + _EPS)
            jac1 = d1 / (a1 + b1 - d1 + _EPS)
            jac2 = d2 / (a2 + b2 - d2 + _EPS)
            dm = (jnp.full((16,), lax.reduce_sum(dice1, axes=(0,)), jnp.float32)
                  + jnp.full((16,), lax.reduce_sum(dice2, axes=(0,)), jnp.float32)
                  ) / float(_NC)
            jm = (jnp.full((16,), lax.reduce_sum(jac1, axes=(0,)), jnp.float32)
                  + jnp.full((16,), lax.reduce_sum(jac2, axes=(0,)), jnp.float32)
                  ) / float(_NC)
            obuf[...] = 1.0 - (dm + jm) / 2.0
            pltpu.sync_copy(obuf, out_hbm)

    return k2


@jax.jit
def _run(pred, true):
    bsz, nc, h, w = pred.shape
    npix = h * w
    pred2 = pred.reshape(bsz, nc, npix)
    true2 = true.reshape(-1).astype(jnp.int32)
    info = plsc.get_sparse_core_info()
    nw = info.num_cores * info.num_subcores
    hists = _make_k1(bsz, npix)(pred2, true2)
    out = _make_k2(nw)(hists)
    return out[0]


def kernel(pred, true):
    return _run(pred, true)
